# Initial kernel scaffold; baseline (speedup 1.0000x reference)
import jax
import jax.numpy as jnp
from jax.experimental import pallas as pl


def kernel(indices_domain_a, indices_domain_b, table):
    raise NotImplementedError("write your pallas kernel here")


# speedup vs baseline: 1.2144x; 1.2144x over previous
import functools

import jax
import jax.numpy as jnp
from jax import lax
from jax.experimental import pallas as pl
from jax.experimental.pallas import tpu as pltpu
from jax.experimental.pallas import tpu_sc as plsc

_EMBED_DIM = 32
_CHUNK = 128
_NBUF = 8


@functools.cache
def _make_gather(n_rows, vocab):
    info = plsc.get_sparse_core_info()
    nc, ns = info.num_cores, info.num_subcores
    nw = nc * ns
    per_w = n_rows // nw
    nchunk = per_w // _CHUNK
    ngroup = nchunk // _NBUF
    assert per_w * nw == n_rows and nchunk * _CHUNK == per_w
    assert ngroup * _NBUF == nchunk

    mesh = plsc.VectorSubcoreMesh(core_axis_name="c", subcore_axis_name="s")

    @functools.partial(
        pl.kernel,
        mesh=mesh,
        compiler_params=pltpu.CompilerParams(use_tc_tiling_on_sc=False),
        out_type=(
            jax.ShapeDtypeStruct((n_rows, _EMBED_DIM), jnp.float32),
            jax.ShapeDtypeStruct((n_rows, _EMBED_DIM), jnp.float32),
        ),
        scratch_types=[
            pltpu.VMEM((nchunk, _CHUNK), jnp.int32),
            pltpu.VMEM((nchunk, _CHUNK), jnp.int32),
            pltpu.VMEM((_NBUF, _CHUNK, _EMBED_DIM), jnp.float32),
            pltpu.SemaphoreType.DMA,
        ],
    )
    def gather2(table_hbm, idx_a_hbm, idx_b_hbm, out_a_hbm, out_b_hbm,
                idx_a_v, idx_b_v, rows_v, sem):
        wid = lax.axis_index("s") * nc + lax.axis_index("c")
        base = wid * per_w
        pltpu.sync_copy(idx_a_hbm.at[wid], idx_a_v)
        pltpu.sync_copy(idx_b_hbm.at[wid], idx_b_v)

        def run_domain(idx_v, out_hbm):
            def group(g, carry):
                c0 = g * _NBUF
                for b in range(_NBUF):
                    pltpu.async_copy(table_hbm.at[idx_v.at[c0 + b]],
                                     rows_v.at[b], sem)
                for b in range(_NBUF):
                    pltpu.make_async_copy(table_hbm.at[idx_v.at[c0 + b]],
                                          rows_v.at[b], sem).wait()
                for b in range(_NBUF):
                    pltpu.sync_copy(
                        rows_v.at[b],
                        out_hbm.at[pl.ds(base + (c0 + b) * _CHUNK, _CHUNK)])
                return carry

            lax.fori_loop(0, ngroup, group, 0)

        run_domain(idx_a_v, out_a_hbm)
        run_domain(idx_b_v, out_b_hbm)

    return gather2


def kernel(indices_domain_a, indices_domain_b, table):
    b, hist = indices_domain_a.shape
    n_rows = b * hist
    info = plsc.get_sparse_core_info()
    nw = info.num_cores * info.num_subcores
    nchunk = n_rows // (nw * _CHUNK)
    idx_a = indices_domain_a.reshape(nw, nchunk, _CHUNK).astype(jnp.int32)
    idx_b = indices_domain_b.reshape(nw, nchunk, _CHUNK).astype(jnp.int32)
    emb_a, emb_b = _make_gather(n_rows, table.shape[0])(table, idx_a, idx_b)
    emb_a = emb_a.reshape(b, hist, _EMBED_DIM)
    emb_b = emb_b.reshape(b, hist, _EMBED_DIM)
    return (indices_domain_a, emb_a, indices_domain_b, emb_b)


# speedup vs baseline: 2.3413x; 1.9280x over previous
import functools

import jax
import jax.numpy as jnp
from jax import lax
from jax.experimental import pallas as pl
from jax.experimental.pallas import tpu as pltpu
from jax.experimental.pallas import tpu_sc as plsc

_D = 32
_BT = 128
_NBUF = 8


@functools.cache
def _make_gather(hist, batch, vocab):
    info = plsc.get_sparse_core_info()
    nc, ns = info.num_cores, info.num_subcores
    nw = nc * ns
    bslice = batch // nw
    ntile = bslice // _BT
    nunit = hist * ntile
    ngroup = nunit // _NBUF
    assert bslice * nw == batch and ntile * _BT == bslice
    assert ngroup * _NBUF == nunit

    mesh = plsc.VectorSubcoreMesh(core_axis_name="c", subcore_axis_name="s")
    out_sds = jax.ShapeDtypeStruct((hist, batch, _D), jnp.float32)
    rows_t = pltpu.VMEM((_BT, _D), jnp.float32)

    @functools.partial(
        pl.kernel,
        mesh=mesh,
        compiler_params=pltpu.CompilerParams(use_tc_tiling_on_sc=False),
        out_type=(out_sds, out_sds),
        scratch_types=(
            [pltpu.VMEM((2, hist, bslice), jnp.int32)]
            + [rows_t] * _NBUF
            + [pltpu.SemaphoreType.DMA((_NBUF,))]
        ),
    )
    def gather2(table_hbm, idx_a_hbm, idx_b_hbm, out_a_hbm, out_b_hbm,
                idx_v, *refs):
        rows = refs[:_NBUF]
        gsem = refs[_NBUF]
        wid = lax.axis_index("s") * nc + lax.axis_index("c")
        b0 = wid * bslice
        pltpu.sync_copy(idx_a_hbm.at[:, pl.ds(b0, bslice)], idx_v.at[0])
        pltpu.sync_copy(idx_b_hbm.at[:, pl.ds(b0, bslice)], idx_v.at[1])

        def run_domain(dom, out_hbm):
            def gather_op(u, b):
                h = u // ntile
                t = u - h * ntile
                return pltpu.make_async_copy(
                    table_hbm.at[idx_v.at[dom, h, pl.ds(t * _BT, _BT)]],
                    rows[b], gsem.at[b])

            def store(u, b):
                h = u // ntile
                t = u - h * ntile
                pltpu.sync_copy(
                    rows[b], out_hbm.at[h, pl.ds(b0 + t * _BT, _BT), :])

            for b in range(_NBUF):
                gather_op(b, b).start()

            def group(g, carry):
                for b in range(_NBUF):
                    u = g * _NBUF + b
                    gather_op(u, b).wait()
                    store(u, b)

                    @pl.when(g < ngroup - 1)
                    def _():
                        gather_op(u + _NBUF, b).start()

                return carry

            lax.fori_loop(0, ngroup, group, 0)

        run_domain(0, out_a_hbm)
        run_domain(1, out_b_hbm)

    return gather2


def kernel(indices_domain_a, indices_domain_b, table):
    batch, hist = indices_domain_a.shape
    idx_at = indices_domain_a.T.astype(jnp.int32)
    idx_bt = indices_domain_b.T.astype(jnp.int32)
    out_a, out_b = _make_gather(hist, batch, table.shape[0])(
        table, idx_at, idx_bt)

    def form(o3):
        return jnp.transpose(o3, (1, 0, 2))

    return (indices_domain_a, form(out_a), indices_domain_b, form(out_b))
